# baseline (speedup 1.0000x reference)
import jax
import jax.numpy as jnp
from jax import lax
from jax.experimental import pallas as pl
from jax.experimental.pallas import tpu as pltpu
from jax.experimental.pallas import tpu_sc as plsc

N = 10000
E = 320000
D = 128
NC = 2
NS = 16
NW = NC * NS
EPW = E // NW
CRAW = 125
CBIG = 128
NCHB = E // CRAW
C = 64
NCH = NCHB * 2
CPW = NCH // NW
NPAD = 10240
RPT = NPAD // NS
L = 16

ROWS_BLK = 1000
GRID_R = N // ROWS_BLK


def _mm_body(x_ref, w_ref, o_ref):
    o_ref[...] = lax.dot_general(x_ref[...], w_ref[...],
                                 (((1,), (1,)), ((), ())),
                                 preferred_element_type=jnp.float32)


def _mm(x, w):
    return pl.pallas_call(
        _mm_body,
        grid=(GRID_R,),
        in_specs=[
            pl.BlockSpec((ROWS_BLK, D), lambda i: (i, 0)),
            pl.BlockSpec((D, D), lambda i: (0, 0)),
        ],
        out_specs=pl.BlockSpec((ROWS_BLK, D), lambda i: (i, 0)),
        out_shape=jax.ShapeDtypeStruct((N, D), jnp.float32),
    )(x, w)


def _inv_cnt_body(c_ref, o_ref):
    s = jnp.sum(c_ref[...], axis=0)
    o_ref[...] = (1.0 / jnp.maximum(s, 1.0))[:, None]


def _inv_cnt(cnts):
    return pl.pallas_call(
        _inv_cnt_body,
        out_shape=jax.ShapeDtypeStruct((NPAD, 1), jnp.float32),
    )(cnts)


def _combine1_body(part_ref, inv_ref, x_ref, wr_ref, b_ref, wl2_ref,
                   h_ref, p2_ref):
    agg = part_ref[0] + part_ref[1]
    mean_lin = agg * inv_ref[...]
    xr = lax.dot_general(x_ref[...], wr_ref[...], (((1,), (1,)), ((), ())),
                         preferred_element_type=jnp.float32)
    h = jnp.maximum(mean_lin + b_ref[...] + xr, 0.0)
    h_ref[...] = h
    p2_ref[...] = lax.dot_general(h, wl2_ref[...], (((1,), (1,)), ((), ())),
                                  preferred_element_type=jnp.float32)


def _combine1(parts, inv, x, wr, b, wl2):
    return pl.pallas_call(
        _combine1_body,
        grid=(GRID_R,),
        in_specs=[
            pl.BlockSpec((NC, ROWS_BLK, D), lambda i: (0, i, 0)),
            pl.BlockSpec((ROWS_BLK, 1), lambda i: (i, 0)),
            pl.BlockSpec((ROWS_BLK, D), lambda i: (i, 0)),
            pl.BlockSpec((D, D), lambda i: (0, 0)),
            pl.BlockSpec((1, D), lambda i: (0, 0)),
            pl.BlockSpec((D, D), lambda i: (0, 0)),
        ],
        out_specs=[
            pl.BlockSpec((ROWS_BLK, D), lambda i: (i, 0)),
            pl.BlockSpec((ROWS_BLK, D), lambda i: (i, 0)),
        ],
        out_shape=[
            jax.ShapeDtypeStruct((N, D), jnp.float32),
            jax.ShapeDtypeStruct((N, D), jnp.float32),
        ],
    )(parts, inv, x, wr, b, wl2)


def _combine2_body(part_ref, inv_ref, h_ref, wr_ref, b_ref, o_ref):
    agg = part_ref[0] + part_ref[1]
    mean_lin = agg * inv_ref[...]
    hr = lax.dot_general(h_ref[...], wr_ref[...], (((1,), (1,)), ((), ())),
                         preferred_element_type=jnp.float32)
    o_ref[...] = mean_lin + b_ref[...] + hr


def _combine2(parts, inv, h, wr, b):
    return pl.pallas_call(
        _combine2_body,
        grid=(GRID_R,),
        in_specs=[
            pl.BlockSpec((NC, ROWS_BLK, D), lambda i: (0, i, 0)),
            pl.BlockSpec((ROWS_BLK, 1), lambda i: (i, 0)),
            pl.BlockSpec((ROWS_BLK, D), lambda i: (i, 0)),
            pl.BlockSpec((D, D), lambda i: (0, 0)),
            pl.BlockSpec((1, D), lambda i: (0, 0)),
        ],
        out_specs=pl.BlockSpec((ROWS_BLK, D), lambda i: (i, 0)),
        out_shape=jax.ShapeDtypeStruct((N, D), jnp.float32),
    )(parts, inv, h, wr, b)


def _sc_agg_body(p_hbm, src2_hbm, dst2_hbm, zero_hbm, out_hbm, cnt_hbm,
                 src_all, dst_all, buf0, buf1, cnt_v, acc_sh, sem0, sem1):
    cid = lax.axis_index("c")
    sid = lax.axis_index("s")
    wid = cid * NS + sid

    pltpu.sync_copy(zero_hbm, acc_sh.at[pl.ds(sid * RPT, RPT)])

    zeros16 = jnp.zeros((L,), jnp.float32)

    def zinit(i, carry):
        cnt_v[pl.ds(i * L, L)] = zeros16
        return carry

    lax.fori_loop(0, NPAD // L, zinit, 0)

    pltpu.sync_copy(src2_hbm.at[pl.ds(wid * CPW, CPW)], src_all)
    pltpu.sync_copy(dst2_hbm.at[pl.ds(wid * CPW, CPW)], dst_all)
    plsc.subcore_barrier()

    bufs = (buf0, buf1)
    sems = (sem0, sem1)
    ones16 = jnp.ones((L,), jnp.float32)

    def gather_start(j, b):
        pltpu.make_async_copy(p_hbm.at[src_all.at[j]], bufs[b], sems[b]).start()

    def gather_wait(b):
        pltpu.make_async_copy(p_hbm.at[src_all.at[0]], bufs[b], sems[b]).wait()

    def scatter(j, b):
        pltpu.sync_copy(bufs[b], acc_sh.at[dst_all.at[j]], add=True)

    def counts(j):
        for k in range(C // L):
            idx = dst_all[j, pl.ds(k * L, L)]
            plsc.addupdate_scatter(cnt_v, [idx], ones16)

    gather_start(0, 0)

    def outer(jo, carry):
        j0 = 2 * jo
        j1 = j0 + 1
        gather_start(j1, 1)
        gather_wait(0)
        gather_start((j0 + 2) % CPW, 0)
        gather_wait(1)
        return carry

    lax.fori_loop(0, CPW // 2, outer, 0)
    gather_wait(0)
    plsc.subcore_barrier()

    pltpu.sync_copy(acc_sh.at[pl.ds(sid * RPT, RPT)],
                    out_hbm.at[cid, pl.ds(sid * RPT, RPT)])
    pltpu.sync_copy(cnt_v, cnt_hbm.at[wid])


def _sc_agg(p, src2, dst2, zeros):
    mesh = plsc.VectorSubcoreMesh(core_axis_name="c", subcore_axis_name="s")
    kern = pl.kernel(
        _sc_agg_body,
        out_type=(
            jax.ShapeDtypeStruct((NC, NPAD, D), jnp.float32),
            jax.ShapeDtypeStruct((NW, NPAD), jnp.float32),
        ),
        mesh=mesh,
        scratch_types=[
            pltpu.VMEM((CPW, C), jnp.int32),
            pltpu.VMEM((CPW, C), jnp.int32),
            pltpu.VMEM((C, D), jnp.float32),
            pltpu.VMEM((C, D), jnp.float32),
            pltpu.VMEM((NPAD,), jnp.float32),
            pltpu.VMEM_SHARED((NPAD, D), jnp.float32),
            pltpu.SemaphoreType.DMA,
            pltpu.SemaphoreType.DMA,
        ],
        compiler_params=pltpu.CompilerParams(needs_layout_passes=False, use_tc_tiling_on_sc=False),
    )
    return kern(p, src2, dst2, zeros)


def kernel(x, edge_index, Wl1, bl1, Wr1, Wl2, bl2, Wr2):
    ei = edge_index.astype(jnp.int32)
    src2 = jnp.concatenate(
        [ei[0].reshape(NCHB, CRAW),
         jnp.zeros((NCHB, CBIG - CRAW), jnp.int32)], axis=1).reshape(NCH, C)
    dst2 = jnp.concatenate(
        [ei[1].reshape(NCHB, CRAW),
         jnp.full((NCHB, CBIG - CRAW), NPAD - 1, jnp.int32)], axis=1).reshape(NCH, C)
    zeros = jnp.zeros((RPT, D), jnp.float32)
    bl1r = bl1.reshape(1, D)
    bl2r = bl2.reshape(1, D)

    p1 = _mm(x, Wl1)
    parts1, cnts = _sc_agg(p1, jnp.zeros_like(src2), dst2, zeros)
    inv = _inv_cnt(cnts)
    h, p2 = _combine1(parts1, inv, x, Wr1, bl1r, Wl2)
    parts2, _ = _sc_agg(p2, jnp.zeros_like(src2), dst2, zeros)
    out = _combine2(parts2, inv, h, Wr2, bl2r)
    return out


# speedup vs baseline: 24.9162x; 24.9162x over previous
import jax
import jax.numpy as jnp
from jax import lax
from jax.experimental import pallas as pl
from jax.experimental.pallas import tpu as pltpu
from jax.experimental.pallas import tpu_sc as plsc

N = 10000
E = 320000
D = 128
NC = 2
NS = 16
NW = NC * NS
EPW = E // NW
CRAW = 125
CBIG = 128
NCHB = E // CRAW
C = 64
NCH = NCHB * 2
CPW = NCH // NW
NPAD = 10240
RPT = NPAD // NS
L = 16

ROWS_BLK = 1000
GRID_R = N // ROWS_BLK


def _mm_body(x_ref, w_ref, o_ref):
    o_ref[...] = lax.dot_general(x_ref[...], w_ref[...],
                                 (((1,), (1,)), ((), ())),
                                 preferred_element_type=jnp.float32)


def _mm(x, w):
    return pl.pallas_call(
        _mm_body,
        grid=(GRID_R,),
        in_specs=[
            pl.BlockSpec((ROWS_BLK, D), lambda i: (i, 0)),
            pl.BlockSpec((D, D), lambda i: (0, 0)),
        ],
        out_specs=pl.BlockSpec((ROWS_BLK, D), lambda i: (i, 0)),
        out_shape=jax.ShapeDtypeStruct((N, D), jnp.float32),
    )(x, w)


def _inv_cnt_body(c_ref, o_ref):
    s = jnp.sum(c_ref[...], axis=0)
    o_ref[...] = (1.0 / jnp.maximum(s, 1.0))[:, None]


def _inv_cnt(cnts):
    return pl.pallas_call(
        _inv_cnt_body,
        out_shape=jax.ShapeDtypeStruct((NPAD, 1), jnp.float32),
    )(cnts)


def _combine1_body(part_ref, inv_ref, x_ref, wr_ref, b_ref, wl2_ref,
                   h_ref, p2_ref):
    agg = part_ref[0] + part_ref[1]
    mean_lin = agg * inv_ref[...]
    xr = lax.dot_general(x_ref[...], wr_ref[...], (((1,), (1,)), ((), ())),
                         preferred_element_type=jnp.float32)
    h = jnp.maximum(mean_lin + b_ref[...] + xr, 0.0)
    h_ref[...] = h
    p2_ref[...] = lax.dot_general(h, wl2_ref[...], (((1,), (1,)), ((), ())),
                                  preferred_element_type=jnp.float32)


def _combine1(parts, inv, x, wr, b, wl2):
    return pl.pallas_call(
        _combine1_body,
        grid=(GRID_R,),
        in_specs=[
            pl.BlockSpec((NC, ROWS_BLK, D), lambda i: (0, i, 0)),
            pl.BlockSpec((ROWS_BLK, 1), lambda i: (i, 0)),
            pl.BlockSpec((ROWS_BLK, D), lambda i: (i, 0)),
            pl.BlockSpec((D, D), lambda i: (0, 0)),
            pl.BlockSpec((1, D), lambda i: (0, 0)),
            pl.BlockSpec((D, D), lambda i: (0, 0)),
        ],
        out_specs=[
            pl.BlockSpec((ROWS_BLK, D), lambda i: (i, 0)),
            pl.BlockSpec((ROWS_BLK, D), lambda i: (i, 0)),
        ],
        out_shape=[
            jax.ShapeDtypeStruct((N, D), jnp.float32),
            jax.ShapeDtypeStruct((N, D), jnp.float32),
        ],
    )(parts, inv, x, wr, b, wl2)


def _combine2_body(part_ref, inv_ref, h_ref, wr_ref, b_ref, o_ref):
    agg = part_ref[0] + part_ref[1]
    mean_lin = agg * inv_ref[...]
    hr = lax.dot_general(h_ref[...], wr_ref[...], (((1,), (1,)), ((), ())),
                         preferred_element_type=jnp.float32)
    o_ref[...] = mean_lin + b_ref[...] + hr


def _combine2(parts, inv, h, wr, b):
    return pl.pallas_call(
        _combine2_body,
        grid=(GRID_R,),
        in_specs=[
            pl.BlockSpec((NC, ROWS_BLK, D), lambda i: (0, i, 0)),
            pl.BlockSpec((ROWS_BLK, 1), lambda i: (i, 0)),
            pl.BlockSpec((ROWS_BLK, D), lambda i: (i, 0)),
            pl.BlockSpec((D, D), lambda i: (0, 0)),
            pl.BlockSpec((1, D), lambda i: (0, 0)),
        ],
        out_specs=pl.BlockSpec((ROWS_BLK, D), lambda i: (i, 0)),
        out_shape=jax.ShapeDtypeStruct((N, D), jnp.float32),
    )(parts, inv, h, wr, b)


CPT = NCH // NS
CROWS = NPAD // DH
AOFF = N
COFF = N + NPAD
SROWS = N + NPAD + CROWS
ZPT = (NPAD + CROWS) // NS


def _sc_agg_body(p_hbm, ei2_hbm, out_hbm, cnt_hbm,
                 src_all, dst_all, buf0, cnt2d, zbuf, idx_v, big_sh):
    cid = lax.axis_index("c")
    sid = lax.axis_index("s")

    zeros16 = jnp.zeros((L,), jnp.float32)

    def zinit(i, carry):
        r = i // (DH // L)
        c = pl.ds((i % (DH // L)) * L, L)
        zbuf[r, c] = zeros16
        cnt2d[r, c] = zeros16
        return carry

    lax.fori_loop(0, CROWS * DH // L, zinit, 0)
    zbase = AOFF + sid * ZPT
    for k in range(ZPT // CROWS):
        pltpu.sync_copy(zbuf, big_sh.at[pl.ds(zbase + k * CROWS, CROWS)])
    pltpu.sync_copy(zbuf.at[pl.ds(0, ZPT % CROWS)],
                    big_sh.at[pl.ds(zbase + (ZPT // CROWS) * CROWS,
                                    ZPT % CROWS)])

    iota16 = lax.iota(jnp.int32, L)
    for k in range(CROWS // L):
        idx_v[pl.ds(k * L, L)] = iota16 + (COFF + k * L)

    pltpu.sync_copy(p_hbm.at[cid, pl.ds(sid * TRows, TRows)],
                    big_sh.at[pl.ds(sid * TRows, TRows)])
    plsc.subcore_barrier()

    ones16 = jnp.ones((L,), jnp.float32)

    def chunk(j, carry):
        jg = sid * CPT + j
        pltpu.sync_copy(ei2_hbm.at[0, jg], src_all.at[0])
        pltpu.sync_copy(ei2_hbm.at[1, jg], dst_all.at[0])
        pltpu.sync_copy(big_sh.at[src_all.at[0]], buf0)
        pltpu.sync_copy(buf0, big_sh.at[dst_all.at[0]], add=True)
        for k in range(C // L):
            d = dst_all[0, pl.ds(k * L, L)] - AOFF
            plsc.addupdate_scatter(cnt2d, [d >> 6, d & 63], ones16)
        return carry

    lax.fori_loop(0, CPT, chunk, 0)

    pltpu.sync_copy(cnt2d, big_sh.at[idx_v], add=True)
    plsc.subcore_barrier()

    pltpu.sync_copy(big_sh.at[pl.ds(AOFF + sid * RPT, RPT)],
                    out_hbm.at[cid, pl.ds(sid * RPT, RPT)])
    pltpu.sync_copy(big_sh.at[pl.ds(COFF, CROWS)], cnt_hbm.at[cid])


def _sc_agg(p, ei2):
    mesh = plsc.VectorSubcoreMesh(core_axis_name="c", subcore_axis_name="s")
    kern = pl.kernel(
        _sc_agg_body,
        out_type=(
            jax.ShapeDtypeStruct((NC, NPAD, DH), jnp.float32),
            jax.ShapeDtypeStruct((NC, CROWS, DH), jnp.float32),
        ),
        mesh=mesh,
        scratch_types=[
            pltpu.VMEM((1, C), jnp.int32),
            pltpu.VMEM((1, C), jnp.int32),
            pltpu.VMEM((C, DH), jnp.float32),
            pltpu.VMEM((CROWS, DH), jnp.float32),
            pltpu.VMEM((CROWS, DH), jnp.float32),
            pltpu.VMEM((CROWS,), jnp.int32),
            pltpu.VMEM_SHARED((SROWS, DH), jnp.float32),
        ],
        compiler_params=pltpu.CompilerParams(needs_layout_passes=False, use_tc_tiling_on_sc=False),
    )
    return kern(p, ei2)


def kernel(x, edge_index, Wl1, bl1, Wr1, Wl2, bl2, Wr2):
    ei = edge_index.astype(jnp.int32)
    src2 = jnp.concatenate(
        [ei[0].reshape(NCHB, CRAW),
         jnp.zeros((NCHB, CBIG - CRAW), jnp.int32)], axis=1).reshape(NCH, C)
    dst2 = jnp.concatenate(
        [ei[1].reshape(NCHB, CRAW),
         jnp.full((NCHB, CBIG - CRAW), NPAD - 1, jnp.int32)], axis=1).reshape(NCH, C)
    zeros = jnp.zeros((RPT, D), jnp.float32)
    bl1r = bl1.reshape(1, D)
    bl2r = bl2.reshape(1, D)

    p1 = _mm(x, Wl1)
    parts1, cnts = _sc_agg(p1, src2, dst2, zeros)
    inv = _inv_cnt(cnts.reshape(NC, NPAD))
    h, p2 = _combine1(parts1, inv, x, Wr1, bl1r, Wl2)
    parts2, _ = _sc_agg(p2, src2, dst2, zeros)
    out = _combine2(parts2, inv, h, Wr2, bl2r)
    return out


# speedup vs baseline: 38.8282x; 1.5584x over previous
import jax
import jax.numpy as jnp
from jax import lax
from jax.experimental import pallas as pl
from jax.experimental.pallas import tpu as pltpu
from jax.experimental.pallas import tpu_sc as plsc

N = 10000
E = 320000
D = 128
NC = 2
NS = 16
NW = NC * NS
EPW = E // NW
C = 80
NCHUNK = EPW // C
NPAD = 10240
RPT = NPAD // NS
L = 16

ROWS_BLK = 1000
GRID_R = N // ROWS_BLK


def _mm_body(x_ref, w_ref, o_ref):
    o_ref[...] = lax.dot_general(x_ref[...], w_ref[...],
                                 (((1,), (1,)), ((), ())),
                                 preferred_element_type=jnp.float32
                                 ).astype(jnp.bfloat16)


def _mm(x, w):
    return pl.pallas_call(
        _mm_body,
        grid=(GRID_R,),
        in_specs=[
            pl.BlockSpec((ROWS_BLK, D), lambda i: (i, 0)),
            pl.BlockSpec((D, D), lambda i: (0, 0)),
        ],
        out_specs=pl.BlockSpec((ROWS_BLK, D), lambda i: (i, 0)),
        out_shape=jax.ShapeDtypeStruct((N, D), jnp.bfloat16),
    )(x, w)


def _inv_cnt_body(c_ref, o_ref):
    s = jnp.sum(c_ref[...], axis=0)
    o_ref[...] = (1.0 / jnp.maximum(s, 1.0))[:, None]


def _inv_cnt(cnts):
    return pl.pallas_call(
        _inv_cnt_body,
        out_shape=jax.ShapeDtypeStruct((NPAD, 1), jnp.float32),
    )(cnts)


def _combine1_body(part_ref, inv_ref, x_ref, wr_ref, b_ref, wl2_ref,
                   h_ref, p2_ref):
    agg = (part_ref[0].astype(jnp.float32)
           + part_ref[1].astype(jnp.float32))
    mean_lin = agg * inv_ref[...]
    xr = lax.dot_general(x_ref[...], wr_ref[...], (((1,), (1,)), ((), ())),
                         preferred_element_type=jnp.float32)
    h = jnp.maximum(mean_lin + b_ref[...] + xr, 0.0)
    h_ref[...] = h
    p2_ref[...] = lax.dot_general(h, wl2_ref[...], (((1,), (1,)), ((), ())),
                                  preferred_element_type=jnp.float32
                                  ).astype(jnp.bfloat16)


def _combine1(parts, inv, x, wr, b, wl2):
    return pl.pallas_call(
        _combine1_body,
        grid=(GRID_R,),
        in_specs=[
            pl.BlockSpec((NC, ROWS_BLK, D), lambda i: (0, i, 0)),
            pl.BlockSpec((ROWS_BLK, 1), lambda i: (i, 0)),
            pl.BlockSpec((ROWS_BLK, D), lambda i: (i, 0)),
            pl.BlockSpec((D, D), lambda i: (0, 0)),
            pl.BlockSpec((1, D), lambda i: (0, 0)),
            pl.BlockSpec((D, D), lambda i: (0, 0)),
        ],
        out_specs=[
            pl.BlockSpec((ROWS_BLK, D), lambda i: (i, 0)),
            pl.BlockSpec((ROWS_BLK, D), lambda i: (i, 0)),
        ],
        out_shape=[
            jax.ShapeDtypeStruct((N, D), jnp.float32),
            jax.ShapeDtypeStruct((N, D), jnp.bfloat16),
        ],
    )(parts, inv, x, wr, b, wl2)


def _combine2_body(part_ref, inv_ref, h_ref, wr_ref, b_ref, o_ref):
    agg = (part_ref[0].astype(jnp.float32)
           + part_ref[1].astype(jnp.float32))
    mean_lin = agg * inv_ref[...]
    hr = lax.dot_general(h_ref[...], wr_ref[...], (((1,), (1,)), ((), ())),
                         preferred_element_type=jnp.float32)
    o_ref[...] = mean_lin + b_ref[...] + hr


def _combine2(parts, inv, h, wr, b):
    return pl.pallas_call(
        _combine2_body,
        grid=(GRID_R,),
        in_specs=[
            pl.BlockSpec((NC, ROWS_BLK, D), lambda i: (0, i, 0)),
            pl.BlockSpec((ROWS_BLK, 1), lambda i: (i, 0)),
            pl.BlockSpec((ROWS_BLK, D), lambda i: (i, 0)),
            pl.BlockSpec((D, D), lambda i: (0, 0)),
            pl.BlockSpec((1, D), lambda i: (0, 0)),
        ],
        out_specs=pl.BlockSpec((ROWS_BLK, D), lambda i: (i, 0)),
        out_shape=jax.ShapeDtypeStruct((N, D), jnp.float32),
    )(parts, inv, h, wr, b)


def _sc_agg_body(p_hbm, src_hbm, dst_hbm, zero_hbm, out_hbm, cnt_hbm,
                 src_v, dst_v, rows_v, cnt_v, acc_sh):
    cid = lax.axis_index("c")
    sid = lax.axis_index("s")
    wid = cid * NS + sid

    pltpu.sync_copy(zero_hbm, acc_sh.at[pl.ds(sid * RPT, RPT)])

    zeros16 = jnp.zeros((L,), jnp.float32)

    def zinit(i, carry):
        cnt_v[pl.ds(i * L, L)] = zeros16
        return carry

    lax.fori_loop(0, NPAD // L, zinit, 0)
    plsc.subcore_barrier()

    ebase = wid * EPW
    ones16 = jnp.ones((L,), jnp.float32)

    def chunk(j, carry):
        base = ebase + j * C
        pltpu.sync_copy(src_hbm.at[pl.ds(base, C)], src_v)
        pltpu.sync_copy(dst_hbm.at[pl.ds(base, C)], dst_v)
        pltpu.sync_copy(p_hbm.at[src_v], rows_v)
        pltpu.sync_copy(rows_v, acc_sh.at[dst_v], add=True)
        for k in range(C // L):
            idx = dst_v[pl.ds(k * L, L)]
            plsc.addupdate_scatter(cnt_v, [idx], ones16)
        return carry

    lax.fori_loop(0, NCHUNK, chunk, 0)
    plsc.subcore_barrier()

    pltpu.sync_copy(acc_sh.at[pl.ds(sid * RPT, RPT)],
                    out_hbm.at[cid, pl.ds(sid * RPT, RPT)])
    pltpu.sync_copy(cnt_v, cnt_hbm.at[wid])


def _sc_agg(p, src, dst, zeros):
    mesh = plsc.VectorSubcoreMesh(core_axis_name="c", subcore_axis_name="s")
    kern = pl.kernel(
        _sc_agg_body,
        out_type=(
            jax.ShapeDtypeStruct((NC, NPAD, D), jnp.bfloat16),
            jax.ShapeDtypeStruct((NW, NPAD), jnp.float32),
        ),
        mesh=mesh,
        scratch_types=[
            pltpu.VMEM((C,), jnp.int32),
            pltpu.VMEM((C,), jnp.int32),
            pltpu.VMEM((C, D), jnp.bfloat16),
            pltpu.VMEM((NPAD,), jnp.float32),
            pltpu.VMEM_SHARED((NPAD, D), jnp.bfloat16),
        ],
        compiler_params=pltpu.CompilerParams(needs_layout_passes=False,
                                             use_tc_tiling_on_sc=False),
    )
    return kern(p, src, dst, zeros)


def kernel(x, edge_index, Wl1, bl1, Wr1, Wl2, bl2, Wr2):
    ei = edge_index.astype(jnp.int32)
    src = ei[0]
    dst = ei[1]
    zeros = jnp.zeros((RPT, D), jnp.bfloat16)
    bl1r = bl1.reshape(1, D)
    bl2r = bl2.reshape(1, D)

    p1 = _mm(x, Wl1)
    parts1, cnts = _sc_agg(p1, src, dst, zeros)
    inv = _inv_cnt(cnts)
    h, p2 = _combine1(parts1, inv, x, Wr1, bl1r, Wl2)
    parts2, _ = _sc_agg(p2, src, dst, zeros)
    out = _combine2(parts2, inv, h, Wr2, bl2r)
    return out


# speedup vs baseline: 59.6836x; 1.5371x over previous
import jax
import jax.numpy as jnp
from jax import lax
from jax.experimental import pallas as pl
from jax.experimental.pallas import tpu as pltpu
from jax.experimental.pallas import tpu_sc as plsc

N = 10000
E = 320000
D = 128
NC = 2
NS = 16
NW = NC * NS
EPW = E // NW
CRAW = 125
CBIG = 128
NCHB = E // CRAW
C = 64
NCH = NCHB * 2
CPT = NCH // NS
NPAD = 10240
RPT = NPAD // NS
L = 16
DH = D // 2
TRows = N // NS
AOFF = N
SROWS = N + NPAD

ROWS_BLK = 1000
GRID_R = N // ROWS_BLK


def _mm_body(x_ref, w_ref, o_ref):
    c = pl.program_id(0)
    mm = lax.dot_general(x_ref[...], w_ref[...], (((1,), (1,)), ((), ())),
                         preferred_element_type=jnp.float32)
    o_ref[...] = jnp.where(c == 0, mm[:, :DH],
                           mm[:, DH:]).astype(jnp.bfloat16)


def _mm(x, w):
    return pl.pallas_call(
        _mm_body,
        grid=(NC, GRID_R),
        in_specs=[
            pl.BlockSpec((ROWS_BLK, D), lambda c, i: (i, 0)),
            pl.BlockSpec((D, D), lambda c, i: (0, 0)),
        ],
        out_specs=pl.BlockSpec((ROWS_BLK, DH),
                               lambda c, i: (c * (N // ROWS_BLK) + i, 0)),
        out_shape=jax.ShapeDtypeStruct((NC * N, DH), jnp.bfloat16),
    )(x, w)


def _inv_cnt_body(c_ref, o_ref):
    s = jnp.sum(c_ref[...], axis=0)
    o_ref[...] = (2.0 / jnp.maximum(s, 2.0))[:, None]


def _inv_cnt(cnts):
    return pl.pallas_call(
        _inv_cnt_body,
        out_shape=jax.ShapeDtypeStruct((NPAD, 1), jnp.float32),
    )(cnts)


def _combine1_body(pa_ref, pb_ref, inv_ref, x_ref, wr_ref, b_ref, wl2_ref,
                   h_ref, p2_ref):
    agg = jnp.concatenate([pa_ref[...], pb_ref[...]],
                          axis=1).astype(jnp.float32)
    mean_lin = agg * inv_ref[...]
    xr = lax.dot_general(x_ref[...], wr_ref[...], (((1,), (1,)), ((), ())),
                         preferred_element_type=jnp.float32)
    h = jnp.maximum(mean_lin + b_ref[...] + xr, 0.0)
    h_ref[...] = h
    p2 = lax.dot_general(h, wl2_ref[...], (((1,), (1,)), ((), ())),
                         preferred_element_type=jnp.float32)
    c = pl.program_id(0)
    p2_ref[...] = jnp.where(c == 0, p2[:, :DH],
                            p2[:, DH:]).astype(jnp.bfloat16)


def _combine1(pa, pb, inv, x, wr, b, wl2):
    return pl.pallas_call(
        _combine1_body,
        grid=(NC, GRID_R),
        in_specs=[
            pl.BlockSpec((ROWS_BLK, DH), lambda c, i: (i, 0)),
            pl.BlockSpec((ROWS_BLK, DH), lambda c, i: (i, 0)),
            pl.BlockSpec((ROWS_BLK, 1), lambda c, i: (i, 0)),
            pl.BlockSpec((ROWS_BLK, D), lambda c, i: (i, 0)),
            pl.BlockSpec((D, D), lambda c, i: (0, 0)),
            pl.BlockSpec((1, D), lambda c, i: (0, 0)),
            pl.BlockSpec((D, D), lambda c, i: (0, 0)),
        ],
        out_specs=[
            pl.BlockSpec((ROWS_BLK, D), lambda c, i: (i, 0)),
            pl.BlockSpec((ROWS_BLK, DH),
                         lambda c, i: (c * (N // ROWS_BLK) + i, 0)),
        ],
        out_shape=[
            jax.ShapeDtypeStruct((N, D), jnp.float32),
            jax.ShapeDtypeStruct((NC * N, DH), jnp.bfloat16),
        ],
    )(pa, pb, inv, x, wr, b, wl2)


def _combine2_body(pa_ref, pb_ref, inv_ref, h_ref, wr_ref, b_ref, o_ref):
    agg = jnp.concatenate([pa_ref[...], pb_ref[...]],
                          axis=1).astype(jnp.float32)
    mean_lin = agg * inv_ref[...]
    hr = lax.dot_general(h_ref[...], wr_ref[...], (((1,), (1,)), ((), ())),
                         preferred_element_type=jnp.float32)
    o_ref[...] = mean_lin + b_ref[...] + hr


def _combine2(pa, pb, inv, h, wr, b):
    return pl.pallas_call(
        _combine2_body,
        grid=(GRID_R,),
        in_specs=[
            pl.BlockSpec((ROWS_BLK, DH), lambda i: (i, 0)),
            pl.BlockSpec((ROWS_BLK, DH), lambda i: (i, 0)),
            pl.BlockSpec((ROWS_BLK, 1), lambda i: (i, 0)),
            pl.BlockSpec((ROWS_BLK, D), lambda i: (i, 0)),
            pl.BlockSpec((D, D), lambda i: (0, 0)),
            pl.BlockSpec((1, D), lambda i: (0, 0)),
        ],
        out_specs=pl.BlockSpec((ROWS_BLK, D), lambda i: (i, 0)),
        out_shape=jax.ShapeDtypeStruct((N, D), jnp.float32),
    )(pa, pb, inv, h, wr, b)


def _sc_agg_body(p_hbm, src2_hbm, dst2_hbm, zero_hbm, out_hbm, cnt_hbm,
                 src_all, dst_all, buf0, cnt_v, big_sh):
    cid = lax.axis_index("c")
    sid = lax.axis_index("s")
    wid = cid * NS + sid

    pltpu.sync_copy(zero_hbm, big_sh.at[pl.ds(AOFF + sid * RPT, RPT)])

    zeros16 = jnp.zeros((L,), jnp.float32)

    def zinit(i, carry):
        cnt_v[pl.ds(i * L, L)] = zeros16
        return carry

    lax.fori_loop(0, NPAD // L, zinit, 0)

    pltpu.sync_copy(p_hbm.at[pl.ds(cid * N + sid * TRows, TRows)],
                    big_sh.at[pl.ds(sid * TRows, TRows)])
    pltpu.sync_copy(src2_hbm.at[pl.ds(sid * CPT, CPT)], src_all)
    pltpu.sync_copy(dst2_hbm.at[pl.ds(sid * CPT, CPT)], dst_all)
    plsc.subcore_barrier()

    ones16 = jnp.ones((L,), jnp.float32)

    def chunk(j, carry):
        pltpu.sync_copy(big_sh.at[src_all.at[j]], buf0)
        pltpu.sync_copy(buf0, big_sh.at[dst_all.at[j]], add=True)
        for k in range(C // L):
            d = dst_all[j, pl.ds(k * L, L)] - AOFF
            plsc.addupdate_scatter(cnt_v, [d], ones16)
        return carry

    lax.fori_loop(0, CPT, chunk, 0)
    plsc.subcore_barrier()

    pltpu.sync_copy(big_sh.at[pl.ds(AOFF + sid * RPT, RPT)],
                    out_hbm.at[pl.ds(cid * NPAD + sid * RPT, RPT)])
    pltpu.sync_copy(cnt_v, cnt_hbm.at[wid])


def _sc_agg(p, src2, dst2, zeros):
    mesh = plsc.VectorSubcoreMesh(core_axis_name="c", subcore_axis_name="s")
    kern = pl.kernel(
        _sc_agg_body,
        out_type=(
            jax.ShapeDtypeStruct((NC * NPAD, DH), jnp.bfloat16),
            jax.ShapeDtypeStruct((NW, NPAD), jnp.float32),
        ),
        mesh=mesh,
        scratch_types=[
            pltpu.VMEM((CPT, C), jnp.int32),
            pltpu.VMEM((CPT, C), jnp.int32),
            pltpu.VMEM((C, DH), jnp.bfloat16),
            pltpu.VMEM((NPAD,), jnp.float32),
            pltpu.VMEM_SHARED((SROWS, DH), jnp.bfloat16),
        ],
        compiler_params=pltpu.CompilerParams(needs_layout_passes=False,
                                             use_tc_tiling_on_sc=False),
    )
    return kern(p, src2, dst2, zeros)


def kernel(x, edge_index, Wl1, bl1, Wr1, Wl2, bl2, Wr2):
    ei = edge_index.astype(jnp.int32)
    src2 = jnp.concatenate(
        [ei[0].reshape(NCHB, CRAW),
         jnp.zeros((NCHB, CBIG - CRAW), jnp.int32)], axis=1).reshape(NCH, C)
    dst2 = jnp.concatenate(
        [ei[1].reshape(NCHB, CRAW) + AOFF,
         jnp.full((NCHB, CBIG - CRAW), AOFF + NPAD - 1, jnp.int32)],
        axis=1).reshape(NCH, C)
    zeros = jnp.zeros((RPT, DH), jnp.bfloat16)
    bl1r = bl1.reshape(1, D)
    bl2r = bl2.reshape(1, D)

    p1 = _mm(x, Wl1)
    parts1, cnts = _sc_agg(p1, src2, dst2, zeros)
    inv = _inv_cnt(cnts)
    h, p2 = _combine1(parts1[:N], parts1[NPAD:NPAD + N], inv, x, Wr1,
                      bl1r, Wl2)
    parts2, _ = _sc_agg(p2, src2, dst2, zeros)
    out = _combine2(parts2[:N], parts2[NPAD:NPAD + N], inv, h, Wr2, bl2r)
    return out
